# baseline (speedup 1.0000x reference)
import functools

import jax
import jax.numpy as jnp
from jax import lax
from jax.experimental import pallas as pl
from jax.experimental.pallas import tpu as pltpu
from jax.experimental.pallas import tpu_sc as plsc

NC = 2
NS = 16
NW = NC * NS
K = 16


@functools.lru_cache(maxsize=None)
def _make_gather(batch, du, dm):
    bpw = batch // NW
    ngroups = bpw // K
    mesh = plsc.VectorSubcoreMesh(core_axis_name="c", subcore_axis_name="s")

    @functools.partial(
        pl.kernel,
        mesh=mesh,
        out_type=[
            jax.ShapeDtypeStruct((batch, du), jnp.float32),
            jax.ShapeDtypeStruct((batch, dm), jnp.float32),
        ],
        scratch_types=[
            pltpu.VMEM((bpw,), jnp.int32),
            pltpu.VMEM((bpw,), jnp.int32),
            pltpu.SemaphoreType.DMA,
        ],
    )
    def gather_k(uid_hbm, mid_hbm, utab_hbm, mtab_hbm, uout_hbm, mout_hbm,
                 uidx_v, midx_v, sem):
        wid = lax.axis_index("s") * NC + lax.axis_index("c")
        base = wid * bpw
        pltpu.sync_copy(uid_hbm.at[pl.ds(base, bpw)], uidx_v)
        pltpu.sync_copy(mid_hbm.at[pl.ds(base, bpw)], midx_v)

        def group(g, carry):
            o = g * K
            uvec = uidx_v[pl.ds(o, K)]
            mvec = midx_v[pl.ds(o, K)]
            copies = []
            for t in range(K):
                copies.append(pltpu.async_copy(
                    utab_hbm.at[pl.ds(uvec[t], 1)],
                    uout_hbm.at[pl.ds(base + o + t, 1)], sem))
                copies.append(pltpu.async_copy(
                    mtab_hbm.at[pl.ds(mvec[t], 1)],
                    mout_hbm.at[pl.ds(base + o + t, 1)], sem))
            for cp in copies:
                cp.wait()
            return carry

        lax.fori_loop(0, ngroups, group, 0)

    return gather_k


def _head_body(u_ref, m_ref, g_ref, gw_ref, gb_ref, fcw_ref, fcb_ref, o_ref):
    d = u_ref.shape[1]
    genre_emb = jnp.dot(g_ref[...], gw_ref[...].T,
                        preferred_element_type=jnp.float32) + gb_ref[...]
    fcw = fcw_ref[...]
    wu = fcw[:, 0:d].T
    wm = fcw[:, d:2 * d].T
    wg = fcw[:, 2 * d:3 * d].T
    o_ref[...] = (
        jnp.dot(u_ref[...], wu, preferred_element_type=jnp.float32)
        + jnp.dot(m_ref[...], wm, preferred_element_type=jnp.float32)
        + jnp.dot(genre_emb, wg, preferred_element_type=jnp.float32)
        + fcb_ref[...]
    )


def kernel(user_id, movie_id, genre_features, user_table, movie_table,
           genre_W, genre_b, fc_W, fc_b):
    batch = user_id.shape[0]
    du = user_table.shape[1]
    dm = movie_table.shape[1]
    bpw = batch // NW

    uid = user_id.astype(jnp.int32)
    mid = movie_id.astype(jnp.int32)

    u_emb, m_emb = _make_gather(batch, du, dm)(uid, mid, user_table,
                                               movie_table)

    blk = 4096
    gd = genre_features.shape[1]
    head = pl.pallas_call(
        _head_body,
        grid=(batch // blk,),
        in_specs=[
            pl.BlockSpec((blk, du), lambda i: (i, 0)),
            pl.BlockSpec((blk, dm), lambda i: (i, 0)),
            pl.BlockSpec((blk, gd), lambda i: (i, 0)),
            pl.BlockSpec((du, gd), lambda i: (0, 0)),
            pl.BlockSpec((1, du), lambda i: (0, 0)),
            pl.BlockSpec(fc_W.shape, lambda i: (0, 0)),
            pl.BlockSpec((1, 1), lambda i: (0, 0)),
        ],
        out_specs=pl.BlockSpec((blk, 1), lambda i: (i, 0)),
        out_shape=jax.ShapeDtypeStruct((batch, 1), jnp.float32),
    )
    return head(u_emb, m_emb, genre_features,
                genre_W, genre_b.reshape(1, -1), fc_W, fc_b.reshape(1, 1))


# speedup vs baseline: 1.5411x; 1.5411x over previous
import functools

import jax
import jax.numpy as jnp
from jax import lax
from jax.experimental import pallas as pl
from jax.experimental.pallas import tpu as pltpu
from jax.experimental.pallas import tpu_sc as plsc

NC = 2
NS = 16
NW = NC * NS
CHUNK = 128
L = 16


@functools.lru_cache(maxsize=None)
def _make_fused(batch, du, dm, gd):
    bpw = batch // NW
    nchunks = bpw // CHUNK
    ngroups = bpw // L
    mesh = plsc.VectorSubcoreMesh(core_axis_name="c", subcore_axis_name="s")
    nw = du + dm + gd

    @functools.partial(
        pl.kernel,
        mesh=mesh,
        compiler_params=pltpu.CompilerParams(use_tc_tiling_on_sc=False, needs_layout_passes=False),
        out_type=jax.ShapeDtypeStruct((batch,), jnp.float32),
        scratch_types=[
            pltpu.VMEM((bpw,), jnp.int32),
            pltpu.VMEM((bpw,), jnp.int32),
            pltpu.VMEM((bpw, du), jnp.float32),
            pltpu.VMEM((bpw, dm), jnp.float32),
            pltpu.VMEM((bpw * gd,), jnp.float32),
            pltpu.VMEM(((nw + 1) * L,), jnp.float32),
            pltpu.VMEM((bpw,), jnp.float32),
            pltpu.SemaphoreType.DMA,
            pltpu.SemaphoreType.DMA,
            pltpu.SemaphoreType.DMA,
        ],
    )
    def fused_k(uid_hbm, mid_hbm, gflat_hbm, wq_hbm, utab_hbm, mtab_hbm,
                out_hbm, uidx_v, midx_v, urows_v, mrows_v, g_v, wq_v, out_v,
                usem, msem, gsem):
        wid = lax.axis_index("s") * NC + lax.axis_index("c")
        base = wid * bpw
        pltpu.sync_copy(uid_hbm.at[pl.ds(base, bpw)], uidx_v)
        pltpu.sync_copy(mid_hbm.at[pl.ds(base, bpw)], midx_v)
        gcp = pltpu.async_copy(gflat_hbm.at[pl.ds(base * gd, bpw * gd)],
                               g_v, gsem)
        copies = []
        for c in range(nchunks):
            copies.append(pltpu.async_copy(
                utab_hbm.at[uidx_v.at[pl.ds(c * CHUNK, CHUNK)]],
                urows_v.at[pl.ds(c * CHUNK, CHUNK)], usem))
            copies.append(pltpu.async_copy(
                mtab_hbm.at[midx_v.at[pl.ds(c * CHUNK, CHUNK)]],
                mrows_v.at[pl.ds(c * CHUNK, CHUNK)], msem))
        pltpu.sync_copy(wq_hbm, wq_v)
        gcp.wait()
        for cp in copies:
            cp.wait()

        lane = lax.iota(jnp.int32, L)
        bias = wq_v[pl.ds(nw * L, L)]

        def group(g, carry):
            row = lane + g * L
            acc = bias
            for j in range(du):
                v = plsc.load_gather(urows_v, [row, jnp.full((L,), j, jnp.int32)])
                acc = acc + v * wq_v[pl.ds(j * L, L)]
            for j in range(dm):
                v = plsc.load_gather(mrows_v, [row, jnp.full((L,), j, jnp.int32)])
                acc = acc + v * wq_v[pl.ds((du + j) * L, L)]
            gbase = row * gd
            for j in range(gd):
                v = plsc.load_gather(g_v, [gbase + j])
                acc = acc + v * wq_v[pl.ds((du + dm + j) * L, L)]
            out_v[pl.ds(g * L, L)] = acc
            return carry

        lax.fori_loop(0, ngroups, group, 0)
        pltpu.sync_copy(out_v, out_hbm.at[pl.ds(base, bpw)])

    return fused_k


def kernel(user_id, movie_id, genre_features, user_table, movie_table,
           genre_W, genre_b, fc_W, fc_b):
    batch = user_id.shape[0]
    du = user_table.shape[1]
    dm = movie_table.shape[1]
    gd = genre_features.shape[1]

    uid = user_id.astype(jnp.int32)
    mid = movie_id.astype(jnp.int32)
    gflat = genre_features.reshape(-1)

    w = fc_W[0]
    wg = w[du + dm:]
    wg_eff = genre_W.T @ wg
    bias = fc_b[0] + genre_b @ wg
    wq = jnp.concatenate([
        jnp.repeat(w[:du + dm], L),
        jnp.repeat(wg_eff, L),
        jnp.full((L,), bias, jnp.float32),
    ])

    out = _make_fused(batch, du, dm, gd)(uid, mid, gflat, wq,
                                         user_table, movie_table)
    return out.reshape(batch, 1)


# speedup vs baseline: 2.0424x; 1.3253x over previous
import functools

import jax
import jax.numpy as jnp
from jax import lax
from jax.experimental import pallas as pl
from jax.experimental.pallas import tpu as pltpu
from jax.experimental.pallas import tpu_sc as plsc

NC = 2
NS = 16
NW = NC * NS
L = 16
CHUNK = 128


def _repack_body(x_ref, o_ref):
    d = x_ref.shape[1]
    o_ref[:, 0:d] = x_ref[...]
    o_ref[:, d:] = jnp.zeros_like(o_ref[:, d:])


@functools.lru_cache(maxsize=None)
def _make_fused(batch, du, dm, gd):
    bpw = batch // NW
    ngroups = bpw // L
    nchunks = bpw // CHUNK
    mesh = plsc.VectorSubcoreMesh(core_axis_name="c", subcore_axis_name="s")
    nw = du + dm + gd

    @functools.partial(
        pl.kernel,
        mesh=mesh,
        compiler_params=pltpu.CompilerParams(needs_layout_passes=False),
        out_type=jax.ShapeDtypeStruct((batch,), jnp.float32),
        scratch_types=[
            pltpu.VMEM((bpw,), jnp.int32),
            pltpu.VMEM((bpw,), jnp.int32),
            pltpu.VMEM((bpw, du), jnp.float32),
            pltpu.VMEM((CHUNK, 128), jnp.float32),
            pltpu.VMEM((bpw * du,), jnp.float32),
            pltpu.VMEM((bpw * gd,), jnp.float32),
            pltpu.VMEM(((nw + 1) * L,), jnp.float32),
            pltpu.VMEM((bpw,), jnp.float32),
            pltpu.SemaphoreType.DMA,
            pltpu.SemaphoreType.DMA,
            pltpu.SemaphoreType.DMA,
        ],
    )
    def fused_k(uid_hbm, mid_hbm, gflat_hbm, wq_hbm, utab_hbm, mpad_hbm,
                out_hbm, uidx_v, midx_v, ustage_v, mstage_v, flat_v, g_v,
                wq_v, out_v, usem, msem, gsem):
        wid = lax.axis_index("s") * NC + lax.axis_index("c")
        base = wid * bpw
        pltpu.sync_copy(uid_hbm.at[pl.ds(base, bpw)], uidx_v)
        pltpu.sync_copy(mid_hbm.at[pl.ds(base, bpw)], midx_v)
        gcp = pltpu.async_copy(gflat_hbm.at[pl.ds(base * gd, bpw * gd)],
                               g_v, gsem)

        def fire(g, carry):
            o = g * L
            uvec = uidx_v[pl.ds(o, L)]
            for t in range(L):
                pltpu.async_copy(
                    utab_hbm.at[pl.ds(uvec[t], 1)],
                    ustage_v.at[pl.ds(o + t, 1)], usem)
            return carry

        lax.fori_loop(0, ngroups, fire, 0)
        pltpu.sync_copy(wq_hbm, wq_v)

        for c in range(nchunks):
            pltpu.async_copy(
                mpad_hbm.at[midx_v.at[pl.ds(c * CHUNK, CHUNK)]],
                mstage_v, msem).wait()

            def mcompact(i, carry):
                fo = (c * CHUNK + i) * dm
                for h in range(dm // L):
                    flat_v[pl.ds(fo + h * L, L)] = \
                        mstage_v[i, pl.ds(h * L, L)]
                return carry

            lax.fori_loop(0, CHUNK, mcompact, 0)

        gcp.wait()
        lane = lax.iota(jnp.int32, L)
        bias = wq_v[pl.ds(nw * L, L)]

        def group1(g, carry):
            row = lane + g * L
            acc = bias
            mb = row * dm
            for j in range(dm):
                v = plsc.load_gather(flat_v, [mb + j])
                acc = acc + v * wq_v[pl.ds((du + j) * L, L)]
            gb = row * gd
            for j in range(gd):
                v = plsc.load_gather(g_v, [gb + j])
                acc = acc + v * wq_v[pl.ds((du + dm + j) * L, L)]
            out_v[pl.ds(g * L, L)] = acc
            return carry

        lax.fori_loop(0, ngroups, group1, 0)

        pltpu.make_async_copy(utab_hbm.at[pl.ds(0, bpw)], ustage_v,
                              usem).wait()

        def ucompact(i, carry):
            fo = i * du
            for h in range(du // L):
                flat_v[pl.ds(fo + h * L, L)] = ustage_v[i, pl.ds(h * L, L)]
            return carry

        lax.fori_loop(0, bpw, ucompact, 0)

        def group2(g, carry):
            row = lane + g * L
            acc = out_v[pl.ds(g * L, L)]
            ub = row * du
            for j in range(du):
                v = plsc.load_gather(flat_v, [ub + j])
                acc = acc + v * wq_v[pl.ds(j * L, L)]
            out_v[pl.ds(g * L, L)] = acc
            return carry

        lax.fori_loop(0, ngroups, group2, 0)
        pltpu.sync_copy(out_v, out_hbm.at[pl.ds(base, bpw)])

    return fused_k


def kernel(user_id, movie_id, genre_features, user_table, movie_table,
           genre_W, genre_b, fc_W, fc_b):
    batch = user_id.shape[0]
    du = user_table.shape[1]
    dm = movie_table.shape[1]
    gd = genre_features.shape[1]
    nm = movie_table.shape[0]

    uid = user_id.astype(jnp.int32)
    mid = movie_id.astype(jnp.int32)
    gflat = genre_features.reshape(-1)

    blk = 5000
    repack = pl.pallas_call(
        _repack_body,
        grid=(nm // blk,),
        in_specs=[pl.BlockSpec((blk, dm), lambda i: (i, 0))],
        out_specs=pl.BlockSpec((blk, 128), lambda i: (i, 0)),
        out_shape=jax.ShapeDtypeStruct((nm, 128), jnp.float32),
    )
    mpad = repack(movie_table)

    w = fc_W[0]
    wg = w[du + dm:]
    wg_eff = genre_W.T @ wg
    bias = fc_b[0] + genre_b @ wg
    wq = jnp.concatenate([
        jnp.repeat(w[:du + dm], L),
        jnp.repeat(wg_eff, L),
        jnp.full((L,), bias, jnp.float32),
    ])

    out = _make_fused(batch, du, dm, gd)(uid, mid, gflat, wq,
                                         user_table, mpad)
    return out.reshape(batch, 1)
